# baseline (speedup 1.0000x reference)
import jax
import jax.numpy as jnp
from jax import lax
from jax.experimental import pallas as pl
from jax.experimental.pallas import tpu as pltpu
from jax.experimental.pallas import tpu_sc as plsc

VOCAB = 1000000
EMBED = 64
B = 4096
L = 200

NC = 2
NS = 16
NW = NC * NS

ROWS_W = B // NW
CHB = 2
N_CHUNKS = ROWS_W // CHB
SPLITS = ((0, 128), (128, 72))
GROUPS = tuple(q * 16 for q in range(12)) + (184,)

SEG_REP = 2048


def _body(seq_hbm, lbl_hbm, tok_hbm, seg_hbm, out_hbm,
          idx0, lbl0, rows0, segr0, idx1, lbl1, rows1, segr1,
          g0, g1, w0, w1):
    wid = lax.axis_index("s") * NC + lax.axis_index("c")
    b_base = wid * ROWS_W
    lanes = lax.iota(jnp.int32, 16)

    bufs = ((idx0, lbl0, rows0, segr0, g0, w0),
            (idx1, lbl1, rows1, segr1, g1, w1))

    def prep(c, bs):
        idx_v, lbl_v, rows_v, segr_v, g, _ = bs
        b0 = b_base + c * CHB
        pltpu.sync_copy(seq_hbm.at[pl.ds(b0, CHB)], idx_v)
        pltpu.sync_copy(lbl_hbm.at[pl.ds(b0, CHB)], lbl_v)
        for j in range(CHB):
            for off in GROUPS:
                sl = pl.ds(off, 16)
                koff = wid * (CHB * L) + j * L + off
                k = (lanes + koff) & (SEG_REP - 1)
                lbl_v[j, sl] = lax.rem(lbl_v[j, sl], 3) + k * 3
        for j in range(CHB):
            for off, ln in SPLITS:
                pltpu.async_copy(tok_hbm.at[idx_v.at[j, pl.ds(off, ln)]],
                                 rows_v.at[j, pl.ds(off, ln)], g)
                pltpu.async_copy(seg_hbm.at[lbl_v.at[j, pl.ds(off, ln)]],
                                 segr_v.at[j, pl.ds(off, ln)], g)

    def finish(c, bs):
        idx_v, lbl_v, rows_v, segr_v, g, w = bs
        pltpu.make_async_copy(out_hbm.at[pl.ds(0, CHB)], rows_v, g).wait()
        pltpu.make_async_copy(out_hbm.at[pl.ds(0, CHB)], segr_v, g).wait()

        def add_row(r, _):
            for j in range(CHB):
                for q in range(EMBED // 16):
                    sl = pl.ds(q * 16, 16)
                    rows_v[j, r, sl] = rows_v[j, r, sl] + segr_v[j, r, sl]
            return 0

        lax.fori_loop(0, L, add_row, 0)
        b0 = b_base + c * CHB
        for j in range(CHB):
            pltpu.async_copy(rows_v.at[j], out_hbm.at[b0 + j], w)

    def drain_w(bs):
        _, _, rows_v, _, _, w = bs
        for j in range(CHB):
            pltpu.make_async_copy(rows_v.at[j], out_hbm.at[0], w).wait()

    prep(0, bufs[0])

    def iter_t(t, _):
        c0 = 2 * t
        c1 = c0 + 1

        @pl.when(t != 0)
        def _():
            drain_w(bufs[1])

        prep(c1, bufs[1])
        finish(c0, bufs[0])

        drain_w(bufs[0])

        @pl.when(c1 + 1 < N_CHUNKS)
        def _():
            prep(c1 + 1, bufs[0])

        finish(c1, bufs[1])
        return 0

    lax.fori_loop(0, N_CHUNKS // 2, iter_t, 0)
    drain_w(bufs[1])


@jax.jit
def _run(sequence, segment_label, token_table, seg_big):
    mesh = plsc.VectorSubcoreMesh(core_axis_name="c", subcore_axis_name="s")
    f = pl.kernel(
        _body,
        out_type=jax.ShapeDtypeStruct((B, L, EMBED), jnp.float32),
        mesh=mesh,
        scratch_types=[
            pltpu.VMEM((CHB, L), jnp.int32),
            pltpu.VMEM((CHB, L), jnp.int32),
            pltpu.VMEM((CHB, L, EMBED), jnp.float32),
            pltpu.VMEM((CHB, L, EMBED), jnp.float32),
            pltpu.VMEM((CHB, L), jnp.int32),
            pltpu.VMEM((CHB, L), jnp.int32),
            pltpu.VMEM((CHB, L, EMBED), jnp.float32),
            pltpu.VMEM((CHB, L, EMBED), jnp.float32),
            pltpu.SemaphoreType.DMA,
            pltpu.SemaphoreType.DMA,
            pltpu.SemaphoreType.DMA,
            pltpu.SemaphoreType.DMA,
        ],
        compiler_params=pltpu.CompilerParams(use_tc_tiling_on_sc=False),
    )
    return f(sequence, segment_label, token_table, seg_big)


def kernel(sequence, segment_label, token_table, segment_table):
    seg_big = jnp.tile(segment_table, (SEG_REP, 1))
    return _run(sequence, segment_label, token_table, seg_big)


# speedup vs baseline: 1.0580x; 1.0580x over previous
import jax
import jax.numpy as jnp
from jax import lax
from jax.experimental import pallas as pl
from jax.experimental.pallas import tpu as pltpu
from jax.experimental.pallas import tpu_sc as plsc

VOCAB = 1000000
EMBED = 64
B = 4096
L = 200

NC = 2
NS = 16
NW = NC * NS

N = B * L
IDX_W = 128
PER_W = N // NW
CH = 256
IR = CH // IDX_W
N_CHUNKS = PER_W // CH
IROWS_W = PER_W // IDX_W

SEG_REP = 2048


def _body(seq_hbm, lbl_hbm, tok_hbm, seg_hbm, out_hbm,
          idx0, lbl0, rows0, segr0, idx1, lbl1, rows1, segr1,
          g0, g1, w0, w1):
    wid = lax.axis_index("s") * NC + lax.axis_index("c")
    row0 = wid * IROWS_W
    lanes = lax.iota(jnp.int32, 16)

    bufs = ((idx0, lbl0, rows0, segr0, g0, w0),
            (idx1, lbl1, rows1, segr1, g1, w1))

    def prep(c, bs):
        idx_v, lbl_v, rows_v, segr_v, g, _ = bs
        ir0 = row0 + c * IR
        pltpu.sync_copy(seq_hbm.at[pl.ds(ir0, IR)], idx_v)
        pltpu.sync_copy(lbl_hbm.at[pl.ds(ir0, IR)], lbl_v)
        for j in range(IR):
            for q in range(IDX_W // 16):
                sl = pl.ds(q * 16, 16)
                koff = wid * CH + j * IDX_W + q * 16
                lbl_v[j, sl] = lbl_v[j, sl] + ((lanes + koff) & (SEG_REP - 1)) * 3
        for j in range(IR):
            pltpu.async_copy(seg_hbm.at[lbl_v.at[j]],
                             rows_v.at[pl.ds(j * IDX_W, IDX_W)], g)
        pltpu.make_async_copy(out_hbm.at[pl.ds(0, CH)], rows_v, g).wait()
        for j in range(IR):
            pltpu.async_copy(tok_hbm.at[idx_v.at[j]],
                             rows_v.at[pl.ds(j * IDX_W, IDX_W)], g, add=True)

    def finish(c, bs):
        idx_v, lbl_v, rows_v, segr_v, g, w = bs
        pltpu.make_async_copy(out_hbm.at[pl.ds(0, CH)], rows_v, g).wait()
        pltpu.async_copy(rows_v, out_hbm.at[pl.ds((row0 + c * IR) * IDX_W, CH)], w)

    def drain_w(bs):
        _, _, rows_v, _, _, w = bs
        pltpu.make_async_copy(rows_v, out_hbm.at[pl.ds(0, CH)], w).wait()

    prep(0, bufs[0])

    def iter_t(t, _):
        c0 = 2 * t
        c1 = c0 + 1

        @pl.when(t != 0)
        def _():
            drain_w(bufs[1])

        prep(c1, bufs[1])
        finish(c0, bufs[0])

        drain_w(bufs[0])

        @pl.when(c1 + 1 < N_CHUNKS)
        def _():
            prep(c1 + 1, bufs[0])

        finish(c1, bufs[1])
        return 0

    lax.fori_loop(0, N_CHUNKS // 2, iter_t, 0)
    drain_w(bufs[1])


@jax.jit
def _run(seq2, lbl2, token_table, seg_big):
    mesh = plsc.VectorSubcoreMesh(core_axis_name="c", subcore_axis_name="s")
    f = pl.kernel(
        _body,
        out_type=jax.ShapeDtypeStruct((N, EMBED), jnp.float32),
        mesh=mesh,
        scratch_types=[
            pltpu.VMEM((IR, IDX_W), jnp.int32),
            pltpu.VMEM((IR, IDX_W), jnp.int32),
            pltpu.VMEM((CH, EMBED), jnp.float32),
            pltpu.VMEM((CH, EMBED), jnp.float32),
            pltpu.VMEM((IR, IDX_W), jnp.int32),
            pltpu.VMEM((IR, IDX_W), jnp.int32),
            pltpu.VMEM((CH, EMBED), jnp.float32),
            pltpu.VMEM((CH, EMBED), jnp.float32),
            pltpu.SemaphoreType.DMA,
            pltpu.SemaphoreType.DMA,
            pltpu.SemaphoreType.DMA,
            pltpu.SemaphoreType.DMA,
        ],
        compiler_params=pltpu.CompilerParams(use_tc_tiling_on_sc=False),
    )
    return f(seq2, lbl2, token_table, seg_big)


def kernel(sequence, segment_label, token_table, segment_table):
    seq2 = sequence.reshape(N // IDX_W, IDX_W)
    lbl2 = segment_label.reshape(N // IDX_W, IDX_W)
    seg_big = jnp.tile(segment_table, (SEG_REP, 1))
    out = _run(seq2, lbl2, token_table, seg_big)
    return out.reshape(B, L, EMBED)


# speedup vs baseline: 1.0882x; 1.0285x over previous
import jax
import jax.numpy as jnp
from jax import lax
from jax.experimental import pallas as pl
from jax.experimental.pallas import tpu as pltpu
from jax.experimental.pallas import tpu_sc as plsc

VOCAB = 1000000
EMBED = 64
B = 4096
L = 200

NC = 2
NS = 16
NW = NC * NS

N = B * L
IDX_W = 128
PER_W = N // NW
CH = 256
IR = CH // IDX_W
N_CHUNKS = PER_W // CH
IROWS_W = PER_W // IDX_W

SEG_REP = 2048


def _body(seq_hbm, lbl_hbm, tok_hbm, seg_hbm, out_hbm,
          idx0, lbl0, rows0, segr0, idx1, lbl1, rows1, segr1,
          g0, g1, w0, w1):
    wid = lax.axis_index("s") * NC + lax.axis_index("c")
    row0 = wid * IROWS_W
    lanes = lax.iota(jnp.int32, 16)

    bufs = ((idx0, lbl0, rows0, segr0, g0, w0),
            (idx1, lbl1, rows1, segr1, g1, w1))

    def prep(c, bs):
        idx_v, lbl_v, rows_v, segr_v, g, _ = bs
        ir0 = row0 + c * IR
        pltpu.sync_copy(seq_hbm.at[pl.ds(ir0, IR)], idx_v)
        pltpu.sync_copy(lbl_hbm.at[pl.ds(ir0, IR)], lbl_v)
        for j in range(IR):
            for q in range(IDX_W // 16):
                sl = pl.ds(q * 16, 16)
                koff = wid * CH + j * IDX_W + q * 16
                lbl_v[j, sl] = lbl_v[j, sl] + ((lanes + koff) & (SEG_REP - 1)) * 3
        for j in range(IR):
            pltpu.async_copy(tok_hbm.at[idx_v.at[j]],
                             rows_v.at[pl.ds(j * IDX_W, IDX_W)], g)
            pltpu.async_copy(seg_hbm.at[lbl_v.at[j]],
                             segr_v.at[pl.ds(j * IDX_W, IDX_W)], g)

    def finish(c, bs):
        idx_v, lbl_v, rows_v, segr_v, g, w = bs
        pltpu.make_async_copy(out_hbm.at[pl.ds(0, CH)], rows_v, g).wait()
        pltpu.make_async_copy(out_hbm.at[pl.ds(0, CH)], rows_v, g).wait()

        def add_row(r, _):
            for q in range(EMBED // 16):
                sl = pl.ds(q * 16, 16)
                rows_v[r, sl] = rows_v[r, sl] + segr_v[r, sl]
            return 0

        lax.fori_loop(0, CH, add_row, 0)
        pltpu.async_copy(rows_v, out_hbm.at[pl.ds((row0 + c * IR) * IDX_W, CH)], w)

    def drain_w(bs):
        _, _, rows_v, _, _, w = bs
        pltpu.make_async_copy(rows_v, out_hbm.at[pl.ds(0, CH)], w).wait()

    prep(0, bufs[0])

    def iter_t(t, _):
        c0 = 2 * t
        c1 = c0 + 1

        @pl.when(t != 0)
        def _():
            drain_w(bufs[1])

        prep(c1, bufs[1])
        finish(c0, bufs[0])

        drain_w(bufs[0])

        @pl.when(c1 + 1 < N_CHUNKS)
        def _():
            prep(c1 + 1, bufs[0])

        finish(c1, bufs[1])
        return 0

    lax.fori_loop(0, N_CHUNKS // 2, iter_t, 0)
    drain_w(bufs[1])


@jax.jit
def _run(seq2, lbl2, token_table, seg_big):
    mesh = plsc.VectorSubcoreMesh(core_axis_name="c", subcore_axis_name="s")
    f = pl.kernel(
        _body,
        out_type=jax.ShapeDtypeStruct((N, EMBED), jnp.float32),
        mesh=mesh,
        scratch_types=[
            pltpu.VMEM((IR, IDX_W), jnp.int32),
            pltpu.VMEM((IR, IDX_W), jnp.int32),
            pltpu.VMEM((CH, EMBED), jnp.float32),
            pltpu.VMEM((CH, EMBED), jnp.float32),
            pltpu.VMEM((IR, IDX_W), jnp.int32),
            pltpu.VMEM((IR, IDX_W), jnp.int32),
            pltpu.VMEM((CH, EMBED), jnp.float32),
            pltpu.VMEM((CH, EMBED), jnp.float32),
            pltpu.SemaphoreType.DMA,
            pltpu.SemaphoreType.DMA,
            pltpu.SemaphoreType.DMA,
            pltpu.SemaphoreType.DMA,
        ],
        compiler_params=pltpu.CompilerParams(use_tc_tiling_on_sc=False),
    )
    return f(seq2, lbl2, token_table, seg_big)


def kernel(sequence, segment_label, token_table, segment_table):
    seq2 = sequence.reshape(N // IDX_W, IDX_W)
    lbl2 = segment_label.reshape(N // IDX_W, IDX_W)
    seg_big = jnp.tile(segment_table, (SEG_REP, 1))
    out = _run(seq2, lbl2, token_table, seg_big)
    return out.reshape(B, L, EMBED)


# speedup vs baseline: 1.1486x; 1.0556x over previous
import jax
import jax.numpy as jnp
from jax import lax
from jax.experimental import pallas as pl
from jax.experimental.pallas import tpu as pltpu
from jax.experimental.pallas import tpu_sc as plsc

VOCAB = 1000000
EMBED = 64
B = 4096
L = 200

NC = 2
NS = 16
NW = NC * NS

N = B * L
IDX_W = 128
PER_W = N // NW
CH = 512
IR = CH // IDX_W
N_CHUNKS = PER_W // CH
IROWS_W = PER_W // IDX_W

SEG_REP = 2048


def _body(seq_hbm, lbl_hbm, tok_hbm, seg_hbm, out_hbm,
          idx0, lbl0, rows0, idx1, lbl1, rows1,
          g0, g1, w0, w1):
    wid = lax.axis_index("s") * NC + lax.axis_index("c")
    row0 = wid * IROWS_W
    lanes = lax.iota(jnp.int32, 16)

    bufs = ((idx0, lbl0, rows0, g0, w0),
            (idx1, lbl1, rows1, g1, w1))

    def prep_seg(c, bs):
        idx_v, lbl_v, rows_v, g, _ = bs
        ir0 = row0 + c * IR
        pltpu.sync_copy(seq_hbm.at[pl.ds(ir0, IR)], idx_v)
        pltpu.sync_copy(lbl_hbm.at[pl.ds(ir0, IR)], lbl_v)
        for j in range(IR):
            for q in range(IDX_W // 16):
                sl = pl.ds(q * 16, 16)
                koff = wid * CH + j * IDX_W + q * 16
                lbl_v[j, sl] = lbl_v[j, sl] + ((lanes + koff) & (SEG_REP - 1)) * 3
        for j in range(IR):
            pltpu.async_copy(seg_hbm.at[lbl_v.at[j]],
                             rows_v.at[pl.ds(j * IDX_W, IDX_W)], g)

    def prep_tok(bs):
        idx_v, lbl_v, rows_v, g, _ = bs
        pltpu.make_async_copy(out_hbm.at[pl.ds(0, CH)], rows_v, g).wait()
        for j in range(IR):
            pltpu.async_copy(tok_hbm.at[idx_v.at[j]],
                             rows_v.at[pl.ds(j * IDX_W, IDX_W)], g, add=True)

    def finish(c, bs):
        idx_v, lbl_v, rows_v, g, w = bs
        pltpu.make_async_copy(out_hbm.at[pl.ds(0, CH)], rows_v, g).wait()
        pltpu.async_copy(rows_v, out_hbm.at[pl.ds((row0 + c * IR) * IDX_W, CH)], w)

    def drain_w(bs):
        rows_v, w = bs[2], bs[4]
        pltpu.make_async_copy(rows_v, out_hbm.at[pl.ds(0, CH)], w).wait()

    prep_seg(0, bufs[0])
    prep_tok(bufs[0])

    def iter_t(t, _):
        c0 = 2 * t
        c1 = c0 + 1
        c2 = c0 + 2

        @pl.when(t != 0)
        def _():
            drain_w(bufs[1])

        prep_seg(c1, bufs[1])
        finish(c0, bufs[0])
        prep_tok(bufs[1])

        drain_w(bufs[0])

        @pl.when(c2 < N_CHUNKS)
        def _():
            prep_seg(c2, bufs[0])

        finish(c1, bufs[1])

        @pl.when(c2 < N_CHUNKS)
        def _():
            prep_tok(bufs[0])

        return 0

    lax.fori_loop(0, N_CHUNKS // 2, iter_t, 0)
    drain_w(bufs[1])


@jax.jit
def _run(seq2, lbl2, token_table, seg_big):
    mesh = plsc.VectorSubcoreMesh(core_axis_name="c", subcore_axis_name="s")
    f = pl.kernel(
        _body,
        out_type=jax.ShapeDtypeStruct((N, EMBED), jnp.float32),
        mesh=mesh,
        scratch_types=[
            pltpu.VMEM((IR, IDX_W), jnp.int32),
            pltpu.VMEM((IR, IDX_W), jnp.int32),
            pltpu.VMEM((CH, EMBED), jnp.float32),
            pltpu.VMEM((IR, IDX_W), jnp.int32),
            pltpu.VMEM((IR, IDX_W), jnp.int32),
            pltpu.VMEM((CH, EMBED), jnp.float32),
            pltpu.SemaphoreType.DMA,
            pltpu.SemaphoreType.DMA,
            pltpu.SemaphoreType.DMA,
            pltpu.SemaphoreType.DMA,
        ],
        compiler_params=pltpu.CompilerParams(use_tc_tiling_on_sc=False),
    )
    return f(seq2, lbl2, token_table, seg_big)


def kernel(sequence, segment_label, token_table, segment_table):
    seq2 = sequence.reshape(N // IDX_W, IDX_W)
    lbl2 = segment_label.reshape(N // IDX_W, IDX_W)
    seg_big = jnp.tile(segment_table, (SEG_REP, 1))
    out = _run(seq2, lbl2, token_table, seg_big)
    return out.reshape(B, L, EMBED)
